# Initial kernel scaffold; baseline (speedup 1.0000x reference)
import jax
import jax.numpy as jnp
from jax.experimental import pallas as pl


def kernel(x, Q, params, edge_index0, edge_index1, edge_index2, edge_index3, pool_map1, pool_map2, pool_map3):
    raise NotImplementedError("write your pallas kernel here")


# speedup vs baseline: 10.1314x; 10.1314x over previous
import functools

import jax
import jax.numpy as jnp
from jax import lax
from jax.experimental import pallas as pl
from jax.experimental.pallas import tpu as pltpu
from jax.experimental.pallas import tpu_sc as plsc

NT = 16
NC = 2
NW = NC * NT
CHUNK = 128
NSINK = 16


def _ru(x, m):
    return (x + m - 1) // m * m


@functools.lru_cache(maxsize=None)
def _build_seg_sum(n_src, w, CH, gather, n_acc):
    rpt = n_acc // NT
    mesh = plsc.VectorSubcoreMesh(core_axis_name="c", subcore_axis_name="s")
    nfull, rem = divmod(rpt, CHUNK)

    def body(*refs):
        if gather:
            feat, src2, dst2, out, idxs_v, idxd_v, rows_v, acc_sh, sem = refs
        else:
            feat, dst2, out, idxd_v, rows_v, acc_sh, sem = refs
        c = lax.axis_index("c")
        s = lax.axis_index("s")
        wid = c * NT + s

        def zrow(i, carry):
            for j in range(w // 16):
                rows_v[i, pl.ds(j * 16, 16)] = jnp.zeros((16,), jnp.float32)
            return carry
        lax.fori_loop(0, CHUNK, zrow, 0)
        lo = s * rpt

        def zacc(i, carry):
            pltpu.sync_copy(rows_v, acc_sh.at[pl.ds(lo + i * CHUNK, CHUNK)])
            return carry
        lax.fori_loop(0, nfull, zacc, 0)
        if rem:
            pltpu.sync_copy(rows_v.at[pl.ds(0, rem)],
                            acc_sh.at[pl.ds(lo + nfull * CHUNK, rem)])
        plsc.subcore_barrier()

        if gather:
            pltpu.sync_copy(src2.at[wid], idxs_v)
        pltpu.sync_copy(dst2.at[wid], idxd_v)

        def main(j, carry):
            if gather:
                pltpu.sync_copy(feat.at[idxs_v.at[j]], rows_v)
            else:
                pltpu.sync_copy(feat.at[pl.ds((wid * CH + j) * CHUNK, CHUNK)],
                                rows_v)
            pltpu.sync_copy(rows_v, acc_sh.at[idxd_v.at[j]], add=True)
            return carry
        lax.fori_loop(0, CH, main, 0)
        plsc.subcore_barrier()

        off = c * n_acc + lo
        pltpu.sync_copy(acc_sh.at[pl.ds(lo, rpt)], out.at[pl.ds(off, rpt)])

    scratch = []
    if gather:
        scratch.append(pltpu.VMEM((CH, CHUNK), jnp.int32))
    scratch += [
        pltpu.VMEM((CH, CHUNK), jnp.int32),
        pltpu.VMEM((CHUNK, w), jnp.float32),
        pltpu.VMEM_SHARED((n_acc, w), jnp.float32),
        pltpu.SemaphoreType.DMA,
    ]
    return pl.kernel(
        body,
        out_type=jax.ShapeDtypeStruct((NC * n_acc, w), jnp.float32),
        mesh=mesh,
        scratch_types=scratch,
        compiler_params=pltpu.CompilerParams(use_tc_tiling_on_sc=False),
    )


def _edge_seg_sum(feat, src, dst, n_out):
    n_src, w = feat.shape
    E = src.shape[0]
    E_pad = _ru(E, CHUNK * NW)
    n_acc = _ru(n_out + NSINK, NT * 8)
    pad = E_pad - E
    fill = jnp.arange(pad, dtype=jnp.int32)
    src_p = jnp.concatenate([src, fill % n_src]).reshape(NW, -1, CHUNK)
    dst_p = jnp.concatenate([dst, n_out + fill % NSINK]).reshape(NW, -1, CHUNK)
    CH = E_pad // (CHUNK * NW)
    out = _build_seg_sum(n_src, w, CH, True, n_acc)(feat, src_p, dst_p)
    return out.reshape(NC, n_acc, w)[:, :n_out]


def _pool_seg_sum(feat, dst, n_out):
    n_src, w = feat.shape
    E_pad = _ru(n_src, CHUNK * NW)
    n_acc = _ru(n_out + NSINK, NT * 8)
    pad = E_pad - n_src
    fill = jnp.arange(pad, dtype=jnp.int32)
    feat_p = jnp.concatenate(
        [feat, jnp.zeros((pad, w), jnp.float32)], axis=0)
    dst_p = jnp.concatenate([dst, n_out + fill % NSINK]).reshape(NW, -1, CHUNK)
    CH = E_pad // (CHUNK * NW)
    out = _build_seg_sum(E_pad, w, CH, False, n_acc)(feat_p, dst_p)
    return out.reshape(NC, n_acc, w)[:, :n_out]


def _mm(x, W):
    def body(x_ref, w_ref, o_ref):
        o_ref[...] = jnp.dot(x_ref[...], w_ref[...],
                             preferred_element_type=jnp.float32)
    return pl.pallas_call(
        body,
        out_shape=jax.ShapeDtypeStruct((x.shape[0], W.shape[1]), jnp.float32),
    )(x, W)


def _sum_bias(S, b, relu):
    def body(s_ref, b_ref, o_ref):
        r = s_ref[0] + s_ref[1] + b_ref[...]
        o_ref[...] = jnp.maximum(r, 0.0) if relu else r
    return pl.pallas_call(
        body,
        out_shape=jax.ShapeDtypeStruct(S.shape[1:], jnp.float32),
    )(S, b.reshape(1, -1))


def _sum_mm_bias(S, W, b, relu):
    def body(s_ref, w_ref, b_ref, o_ref):
        r = jnp.dot(s_ref[0] + s_ref[1], w_ref[...],
                    preferred_element_type=jnp.float32) + b_ref[...]
        o_ref[...] = jnp.maximum(r, 0.0) if relu else r
    return pl.pallas_call(
        body,
        out_shape=jax.ShapeDtypeStruct((S.shape[1], W.shape[1]), jnp.float32),
    )(S, W, b.reshape(1, -1))


def _film_proj(S, Wbg, bbg, xprev, Qprev, Wd, Wqd):
    wx = xprev.shape[1]
    n = S.shape[1]

    def body(s_ref, wbg_ref, bbg_ref, x_ref, q_ref, wd_ref, wqd_ref, o_ref):
        bg = jnp.dot(s_ref[0] + s_ref[1], wbg_ref[...],
                     preferred_element_type=jnp.float32) + bbg_ref[...]
        xf = x_ref[...] * bg[:, :wx] + bg[:, wx:]
        a = jnp.dot(xf, wd_ref[...], preferred_element_type=jnp.float32)
        bq = jnp.dot(q_ref[...], wqd_ref[...],
                     preferred_element_type=jnp.float32)
        o_ref[...] = jnp.concatenate([a, bq], axis=1)

    return pl.pallas_call(
        body,
        out_shape=jax.ShapeDtypeStruct(
            (n, Wd.shape[1] + Wqd.shape[1]), jnp.float32),
    )(S, Wbg, bbg.reshape(1, -1), xprev, Qprev, Wd, Wqd)


def _film_post(S, Wbg, bbg, xprev, Wp):
    wx = xprev.shape[1]

    def body(s_ref, wbg_ref, bbg_ref, x_ref, wp_ref, o_ref):
        bg = jnp.dot(s_ref[0] + s_ref[1], wbg_ref[...],
                     preferred_element_type=jnp.float32) + bbg_ref[...]
        xf = x_ref[...] * bg[:, :wx] + bg[:, wx:]
        o_ref[...] = jnp.dot(xf, wp_ref[...],
                             preferred_element_type=jnp.float32)

    return pl.pallas_call(
        body,
        out_shape=jax.ShapeDtypeStruct((S.shape[1], Wp.shape[1]), jnp.float32),
    )(S, Wbg, bbg.reshape(1, -1), xprev, Wp)


def _blockdiag(A, B):
    a0, a1 = A.shape
    b0, b1 = B.shape
    top = jnp.concatenate([A, jnp.zeros((a0, b1), jnp.float32)], axis=1)
    bot = jnp.concatenate([jnp.zeros((b0, a1), jnp.float32), B], axis=1)
    return jnp.concatenate([top, bot], axis=0)


def _pad_w(W, rows, cols):
    return jnp.pad(W, ((0, rows - W.shape[0]), (0, cols - W.shape[1])))


def kernel(x, Q, params, edge_index0, edge_index1, edge_index2, edge_index3,
           pool_map1, pool_map2, pool_map3):
    p = params
    N0, NL1 = x.shape[0], 2500
    NL2, NL3 = 625, 156

    feat0 = _mm(jnp.concatenate([x, Q], axis=1),
                _blockdiag(p['pre'][0], p['qpre'][0]))
    S0 = _edge_seg_sum(feat0, edge_index0[0], edge_index0[1], N0)
    b01 = jnp.concatenate([p['pre'][1], p['qpre'][1]])
    u1 = _sum_bias(S0, b01, relu=True)

    P1 = _pool_seg_sum(u1, pool_map1, NL1)
    v1 = _sum_mm_bias(P1, _blockdiag(p['d1'][0], p['qd1'][0]),
                      jnp.concatenate([p['d1'][1], p['qd1'][1]]), relu=False)
    x2, Q2 = v1[:, :64], v1[:, 64:]

    S = _edge_seg_sum(Q2, edge_index1[0], edge_index1[1], NL1)
    Q3 = _sum_mm_bias(S, p['ql1'][0], p['ql1'][1], relu=True)
    S = _edge_seg_sum(Q3, edge_index1[0], edge_index1[1], NL1)
    t1 = _sum_mm_bias(S, p['qp1a'][0], p['qp1a'][1], relu=True)
    S = _edge_seg_sum(t1, edge_index1[0], edge_index1[1], NL1)
    u2 = _film_proj(S, p['qp1b'][0], p['qp1b'][1], x2, Q3,
                    p['d2'][0], _pad_w(p['qd2'][0], 16, 16))

    P2 = _pool_seg_sum(u2, pool_map2, NL2)
    b2 = jnp.concatenate([p['d2'][1], p['qd2'][1],
                          jnp.zeros((8,), jnp.float32)])
    v2 = _sum_bias(P2, b2, relu=False)
    x3, Q4p = v2[:, :32], v2[:, 32:]

    S = _edge_seg_sum(Q4p, edge_index2[0], edge_index2[1], NL2)
    Q5p = _sum_mm_bias(S, _pad_w(p['ql2'][0], 16, 16),
                       jnp.pad(p['ql2'][1], (0, 8)), relu=True)
    S = _edge_seg_sum(Q5p, edge_index2[0], edge_index2[1], NL2)
    t2 = _sum_mm_bias(S, _pad_w(p['qp2a'][0], 16, 32),
                      p['qp2a'][1], relu=True)
    S = _edge_seg_sum(t2, edge_index2[0], edge_index2[1], NL2)
    u3 = _film_proj(S, p['qp2b'][0], p['qp2b'][1], x3, Q5p,
                    p['d3'][0], _pad_w(p['qd3'][0], 16, 16))

    P3 = _pool_seg_sum(u3, pool_map3, NL3)
    b3 = jnp.concatenate([p['d3'][1], p['qd3'][1],
                          jnp.zeros((8,), jnp.float32)])
    v3 = _sum_bias(P3, b3, relu=False)
    x4, Q6p = v3[:, :32], v3[:, 32:]

    S = _edge_seg_sum(Q6p, edge_index3[0], edge_index3[1], NL3)
    Q7p = _sum_mm_bias(S, _pad_w(p['ql3'][0], 16, 16),
                       jnp.pad(p['ql3'][1], (0, 8)), relu=False)
    S = _edge_seg_sum(Q7p, edge_index3[0], edge_index3[1], NL3)
    t3 = _sum_mm_bias(S, _pad_w(p['qp3a'][0], 16, 32),
                      p['qp3a'][1], relu=True)
    S = _edge_seg_sum(t3, edge_index3[0], edge_index3[1], NL3)
    xpost = _film_post(S, p['qp3b'][0], p['qp3b'][1], x4, p['post'][0])
    S = _edge_seg_sum(xpost, edge_index3[0], edge_index3[1], NL3)
    xout = _sum_bias(S, p['post'][1], relu=False)

    return jnp.concatenate([xout, Q7p[:, :8]], axis=1)


# speedup vs baseline: 11.5993x; 1.1449x over previous
import functools

import jax
import jax.numpy as jnp
from jax import lax
from jax.experimental import pallas as pl
from jax.experimental.pallas import tpu as pltpu
from jax.experimental.pallas import tpu_sc as plsc

NT = 16
NC = 2
NW = NC * NT
CHUNK = 128
NSINK = 16


def _ru(x, m):
    return (x + m - 1) // m * m


def _pick_k(ch_raw, w):
    cap = max(1, min(4, (128 * 1024) // (CHUNK * w * 4)))
    cap = min(cap, ch_raw)
    k = min(range(1, cap + 1),
            key=lambda q: (-(-ch_raw // q), -(-ch_raw // q) * q))
    return k, -(-ch_raw // k) * k


@functools.lru_cache(maxsize=None)
def _build_seg_sum(n_src, w, CH, K, gather, n_acc):
    rpt = n_acc // NT
    mesh = plsc.VectorSubcoreMesh(core_axis_name="c", subcore_axis_name="s")
    nfull, rem = divmod(rpt, CHUNK)

    def body(*refs):
        if gather:
            feat, src2, dst2, out, idxs_v, idxd_v, rows_v, acc_sh, sem = refs
        else:
            feat, dst2, out, idxd_v, rows_v, acc_sh, sem = refs
        c = lax.axis_index("c")
        s = lax.axis_index("s")
        wid = c * NT + s

        def zrow(i, carry):
            for j in range(w // 16):
                rows_v[i, pl.ds(j * 16, 16)] = jnp.zeros((16,), jnp.float32)
            return carry
        lax.fori_loop(0, CHUNK, zrow, 0)
        lo = s * rpt

        def zacc(i, carry):
            pltpu.sync_copy(rows_v.at[pl.ds(0, CHUNK)],
                            acc_sh.at[pl.ds(lo + i * CHUNK, CHUNK)])
            return carry
        lax.fori_loop(0, nfull, zacc, 0)
        if rem:
            pltpu.sync_copy(rows_v.at[pl.ds(0, rem)],
                            acc_sh.at[pl.ds(lo + nfull * CHUNK, rem)])
        plsc.subcore_barrier()

        if gather:
            pltpu.sync_copy(src2.at[wid], idxs_v)
        pltpu.sync_copy(dst2.at[wid], idxd_v)

        def main(i, carry):
            base = i * K
            descs = []
            for t in range(K):
                dst_rows = rows_v.at[pl.ds(t * CHUNK, CHUNK)]
                if gather:
                    d = pltpu.async_copy(feat.at[idxs_v.at[base + t]],
                                         dst_rows, sem)
                else:
                    d = pltpu.async_copy(
                        feat.at[pl.ds((wid * CH + base + t) * CHUNK, CHUNK)],
                        dst_rows, sem)
                descs.append(d)
            for d in descs:
                d.wait()
            descs = [pltpu.async_copy(rows_v.at[pl.ds(t * CHUNK, CHUNK)],
                                      acc_sh.at[idxd_v.at[base + t]], sem,
                                      add=True)
                     for t in range(K)]
            for d in descs:
                d.wait()
            return carry
        lax.fori_loop(0, CH // K, main, 0)
        plsc.subcore_barrier()

        off = c * n_acc + lo
        pltpu.sync_copy(acc_sh.at[pl.ds(lo, rpt)], out.at[pl.ds(off, rpt)])

    scratch = []
    if gather:
        scratch.append(pltpu.VMEM((CH, CHUNK), jnp.int32))
    scratch += [
        pltpu.VMEM((CH, CHUNK), jnp.int32),
        pltpu.VMEM((K * CHUNK, w), jnp.float32),
        pltpu.VMEM_SHARED((n_acc, w), jnp.float32),
        pltpu.SemaphoreType.DMA,
    ]
    return pl.kernel(
        body,
        out_type=jax.ShapeDtypeStruct((NC * n_acc, w), jnp.float32),
        mesh=mesh,
        scratch_types=scratch,
        compiler_params=pltpu.CompilerParams(use_tc_tiling_on_sc=False),
    )


def _edge_seg_sum(feat, src, dst, n_out):
    n_src, w = feat.shape
    E = src.shape[0]
    K, CH = _pick_k(-(-E // (CHUNK * NW)), w)
    E_pad = CH * CHUNK * NW
    n_acc = _ru(n_out + NSINK, NT * 8)
    pad = E_pad - E
    fill = jnp.arange(pad, dtype=jnp.int32)
    src_p = jnp.concatenate([src, fill % n_src]).reshape(NW, -1, CHUNK)
    dst_p = jnp.concatenate([dst, n_out + fill % NSINK]).reshape(NW, -1, CHUNK)
    out = _build_seg_sum(n_src, w, CH, K, True, n_acc)(feat, src_p, dst_p)
    return out.reshape(NC, n_acc, w)[:, :n_out]


def _pool_seg_sum(feat, dst, n_out):
    n_src, w = feat.shape
    K, CH = _pick_k(-(-n_src // (CHUNK * NW)), w)
    E_pad = CH * CHUNK * NW
    n_acc = _ru(n_out + NSINK, NT * 8)
    pad = E_pad - n_src
    fill = jnp.arange(pad, dtype=jnp.int32)
    feat_p = jnp.concatenate(
        [feat, jnp.zeros((pad, w), jnp.float32)], axis=0)
    dst_p = jnp.concatenate([dst, n_out + fill % NSINK]).reshape(NW, -1, CHUNK)
    out = _build_seg_sum(E_pad, w, CH, K, False, n_acc)(feat_p, dst_p)
    return out.reshape(NC, n_acc, w)[:, :n_out]


def _mm(x, W):
    def body(x_ref, w_ref, o_ref):
        o_ref[...] = jnp.dot(x_ref[...], w_ref[...],
                             preferred_element_type=jnp.float32)
    return pl.pallas_call(
        body,
        out_shape=jax.ShapeDtypeStruct((x.shape[0], W.shape[1]), jnp.float32),
    )(x, W)


def _sum_bias(S, b, relu):
    def body(s_ref, b_ref, o_ref):
        r = s_ref[0] + s_ref[1] + b_ref[...]
        o_ref[...] = jnp.maximum(r, 0.0) if relu else r
    return pl.pallas_call(
        body,
        out_shape=jax.ShapeDtypeStruct(S.shape[1:], jnp.float32),
    )(S, b.reshape(1, -1))


def _sum_mm_bias(S, W, b, relu):
    def body(s_ref, w_ref, b_ref, o_ref):
        r = jnp.dot(s_ref[0] + s_ref[1], w_ref[...],
                    preferred_element_type=jnp.float32) + b_ref[...]
        o_ref[...] = jnp.maximum(r, 0.0) if relu else r
    return pl.pallas_call(
        body,
        out_shape=jax.ShapeDtypeStruct((S.shape[1], W.shape[1]), jnp.float32),
    )(S, W, b.reshape(1, -1))


def _film_proj(S, Wbg, bbg, xprev, Qprev, Wd, Wqd):
    wx = xprev.shape[1]
    n = S.shape[1]

    def body(s_ref, wbg_ref, bbg_ref, x_ref, q_ref, wd_ref, wqd_ref, o_ref):
        bg = jnp.dot(s_ref[0] + s_ref[1], wbg_ref[...],
                     preferred_element_type=jnp.float32) + bbg_ref[...]
        xf = x_ref[...] * bg[:, :wx] + bg[:, wx:]
        a = jnp.dot(xf, wd_ref[...], preferred_element_type=jnp.float32)
        bq = jnp.dot(q_ref[...], wqd_ref[...],
                     preferred_element_type=jnp.float32)
        o_ref[...] = jnp.concatenate([a, bq], axis=1)

    return pl.pallas_call(
        body,
        out_shape=jax.ShapeDtypeStruct(
            (n, Wd.shape[1] + Wqd.shape[1]), jnp.float32),
    )(S, Wbg, bbg.reshape(1, -1), xprev, Qprev, Wd, Wqd)


def _film_post(S, Wbg, bbg, xprev, Wp):
    wx = xprev.shape[1]

    def body(s_ref, wbg_ref, bbg_ref, x_ref, wp_ref, o_ref):
        bg = jnp.dot(s_ref[0] + s_ref[1], wbg_ref[...],
                     preferred_element_type=jnp.float32) + bbg_ref[...]
        xf = x_ref[...] * bg[:, :wx] + bg[:, wx:]
        o_ref[...] = jnp.dot(xf, wp_ref[...],
                             preferred_element_type=jnp.float32)

    return pl.pallas_call(
        body,
        out_shape=jax.ShapeDtypeStruct((S.shape[1], Wp.shape[1]), jnp.float32),
    )(S, Wbg, bbg.reshape(1, -1), xprev, Wp)


def _blockdiag(A, B):
    a0, a1 = A.shape
    b0, b1 = B.shape
    top = jnp.concatenate([A, jnp.zeros((a0, b1), jnp.float32)], axis=1)
    bot = jnp.concatenate([jnp.zeros((b0, a1), jnp.float32), B], axis=1)
    return jnp.concatenate([top, bot], axis=0)


def _pad_w(W, rows, cols):
    return jnp.pad(W, ((0, rows - W.shape[0]), (0, cols - W.shape[1])))


def kernel(x, Q, params, edge_index0, edge_index1, edge_index2, edge_index3,
           pool_map1, pool_map2, pool_map3):
    p = params
    N0, NL1 = x.shape[0], 2500
    NL2, NL3 = 625, 156

    feat0 = _mm(jnp.concatenate([x, Q], axis=1),
                _blockdiag(p['pre'][0], p['qpre'][0]))
    S0 = _edge_seg_sum(feat0, edge_index0[0], edge_index0[1], N0)
    b01 = jnp.concatenate([p['pre'][1], p['qpre'][1]])
    u1 = _sum_bias(S0, b01, relu=True)

    P1 = _pool_seg_sum(u1, pool_map1, NL1)
    v1 = _sum_mm_bias(P1, _blockdiag(p['d1'][0], p['qd1'][0]),
                      jnp.concatenate([p['d1'][1], p['qd1'][1]]), relu=False)
    x2, Q2 = v1[:, :64], v1[:, 64:]

    S = _edge_seg_sum(Q2, edge_index1[0], edge_index1[1], NL1)
    Q3 = _sum_mm_bias(S, p['ql1'][0], p['ql1'][1], relu=True)
    S = _edge_seg_sum(Q3, edge_index1[0], edge_index1[1], NL1)
    t1 = _sum_mm_bias(S, p['qp1a'][0], p['qp1a'][1], relu=True)
    S = _edge_seg_sum(t1, edge_index1[0], edge_index1[1], NL1)
    u2 = _film_proj(S, p['qp1b'][0], p['qp1b'][1], x2, Q3,
                    p['d2'][0], _pad_w(p['qd2'][0], 16, 16))

    P2 = _pool_seg_sum(u2, pool_map2, NL2)
    b2 = jnp.concatenate([p['d2'][1], p['qd2'][1],
                          jnp.zeros((8,), jnp.float32)])
    v2 = _sum_bias(P2, b2, relu=False)
    x3, Q4p = v2[:, :32], v2[:, 32:]

    S = _edge_seg_sum(Q4p, edge_index2[0], edge_index2[1], NL2)
    Q5p = _sum_mm_bias(S, _pad_w(p['ql2'][0], 16, 16),
                       jnp.pad(p['ql2'][1], (0, 8)), relu=True)
    S = _edge_seg_sum(Q5p, edge_index2[0], edge_index2[1], NL2)
    t2 = _sum_mm_bias(S, _pad_w(p['qp2a'][0], 16, 32),
                      p['qp2a'][1], relu=True)
    S = _edge_seg_sum(t2, edge_index2[0], edge_index2[1], NL2)
    u3 = _film_proj(S, p['qp2b'][0], p['qp2b'][1], x3, Q5p,
                    p['d3'][0], _pad_w(p['qd3'][0], 16, 16))

    P3 = _pool_seg_sum(u3, pool_map3, NL3)
    b3 = jnp.concatenate([p['d3'][1], p['qd3'][1],
                          jnp.zeros((8,), jnp.float32)])
    v3 = _sum_bias(P3, b3, relu=False)
    x4, Q6p = v3[:, :32], v3[:, 32:]

    S = _edge_seg_sum(Q6p, edge_index3[0], edge_index3[1], NL3)
    Q7p = _sum_mm_bias(S, _pad_w(p['ql3'][0], 16, 16),
                       jnp.pad(p['ql3'][1], (0, 8)), relu=False)
    S = _edge_seg_sum(Q7p, edge_index3[0], edge_index3[1], NL3)
    t3 = _sum_mm_bias(S, _pad_w(p['qp3a'][0], 16, 32),
                      p['qp3a'][1], relu=True)
    S = _edge_seg_sum(t3, edge_index3[0], edge_index3[1], NL3)
    xpost = _film_post(S, p['qp3b'][0], p['qp3b'][1], x4, p['post'][0])
    S = _edge_seg_sum(xpost, edge_index3[0], edge_index3[1], NL3)
    xout = _sum_bias(S, p['post'][1], relu=False)

    return jnp.concatenate([xout, Q7p[:, :8]], axis=1)
